# baseline (speedup 1.0000x reference)
import functools

import jax
import jax.numpy as jnp
from jax import lax
from jax.experimental import pallas as pl
from jax.experimental.pallas import tpu as pltpu
from jax.experimental.pallas import tpu_sc as plsc

_BATCH = 128
_VOCAB = 100000
_NC = 2
_NS = 16
_NW = _NC * _NS
_RPW = _BATCH // _NW
_CHUNK = 10000
_CPR = _VOCAB // _CHUNK
_NCHUNKS = _RPW * _CPR
_LANES = 16


def _lane_gather(x, idx):
    return lax.gather(
        x,
        idx[:, None],
        dimension_numbers=lax.GatherDimensionNumbers(
            offset_dims=(), collapsed_slice_dims=(0,), start_index_map=(0,)),
        slice_sizes=(1,),
        mode=lax.GatherScatterMode.PROMISE_IN_BOUNDS,
    )


def _sc_argmax_body(x_hbm, out_hbm, buf0, buf1, res_v, sem0, sem1):
    wid = lax.axis_index("s") * _NC + lax.axis_index("c")
    base = wid * (_RPW * _VOCAB)
    bufs = (buf0, buf1)
    sems = (sem0, sem1)

    def start(g, b):
        pltpu.make_async_copy(
            x_hbm.at[pl.ds(base + g * _CHUNK, _CHUNK)],
            bufs[b],
            sems[b],
        ).start()

    start(0, 0)
    start(1, 1)

    lane = lax.iota(jnp.int32, _LANES)
    res = jnp.zeros((_LANES,), jnp.int32)

    for r in range(_RPW):
        m0 = jnp.full((_LANES,), -jnp.inf, jnp.float32)
        mi0 = jnp.zeros((_LANES,), jnp.int32)

        @pl.loop(0, _CPR, init_carry=(m0, mi0, lane), step=2)
        def chunk_loop(c, carry):
            for b in range(2):
                g = r * _CPR + c + b
                pltpu.make_async_copy(
                    x_hbm.at[pl.ds(base, _CHUNK)], bufs[b], sems[b]
                ).wait()

                @pl.loop(0, _CHUNK, init_carry=carry, step=_LANES, unroll=5)
                def inner(off, ic):
                    m, mi, idx = ic
                    v = bufs[b][pl.ds(off, _LANES)]
                    p = v > m
                    return (
                        jnp.where(p, v, m),
                        jnp.where(p, idx, mi),
                        idx + _LANES,
                    )

                carry = inner

                @pl.when(g + 2 < _NCHUNKS)
                def _():
                    start(g + 2, b)

            return carry

        m, mi, _ = chunk_loop
        for shift in (8, 4, 2, 1):
            perm = lane ^ shift
            ov = _lane_gather(m, perm)
            oi = _lane_gather(mi, perm)
            p = (ov > m) | ((ov == m) & (oi < mi))
            m = jnp.where(p, ov, m)
            mi = jnp.where(p, oi, mi)
        res = jnp.where(lane == r, mi, res)

    res_v[...] = res
    pltpu.sync_copy(res_v, out_hbm.at[wid])


@jax.jit
def _sc_argmax(flat):
    mesh = plsc.VectorSubcoreMesh(
        core_axis_name="c", subcore_axis_name="s",
        num_cores=_NC, num_subcores=_NS)
    f = pl.kernel(
        _sc_argmax_body,
        out_type=jax.ShapeDtypeStruct((_NW, _LANES), jnp.int32),
        mesh=mesh,
        scratch_types=[
            pltpu.VMEM((_CHUNK,), jnp.float32),
            pltpu.VMEM((_CHUNK,), jnp.float32),
            pltpu.VMEM((_LANES,), jnp.int32),
            pltpu.SemaphoreType.DMA,
            pltpu.SemaphoreType.DMA,
        ],
    )
    return f(flat)


def kernel(logits):
    assert logits.shape == (_BATCH, _VOCAB)
    padded = _sc_argmax(logits.reshape(-1))
    return padded[:, :_RPW].reshape(_BATCH)


# speedup vs baseline: 1.0226x; 1.0226x over previous
import functools

import jax
import jax.numpy as jnp
from jax import lax
from jax.experimental import pallas as pl
from jax.experimental.pallas import tpu as pltpu
from jax.experimental.pallas import tpu_sc as plsc

_BATCH = 128
_VOCAB = 100000
_NC = 2
_NS = 16
_NW = _NC * _NS
_RPW = _BATCH // _NW
_CHUNK = 10000
_CPR = _VOCAB // _CHUNK
_NCHUNKS = _RPW * _CPR
_LANES = 16
_NCHAIN = 5


def _lane_gather(x, idx):
    return lax.gather(
        x,
        idx[:, None],
        dimension_numbers=lax.GatherDimensionNumbers(
            offset_dims=(), collapsed_slice_dims=(0,), start_index_map=(0,)),
        slice_sizes=(1,),
        mode=lax.GatherScatterMode.PROMISE_IN_BOUNDS,
    )


def _sc_argmax_body(x_hbm, out_hbm, buf0, buf1, res_v, sem0, sem1):
    wid = lax.axis_index("s") * _NC + lax.axis_index("c")
    base = wid * (_RPW * _VOCAB)
    bufs = (buf0, buf1)
    sems = (sem0, sem1)

    def start(g, b):
        pltpu.make_async_copy(
            x_hbm.at[pl.ds(base + g * _CHUNK, _CHUNK)],
            bufs[b],
            sems[b],
        ).start()

    start(0, 0)
    start(1, 1)

    lane = lax.iota(jnp.int32, _LANES)
    res = jnp.zeros((_LANES,), jnp.int32)
    neg_inf = jnp.full((_LANES,), -jnp.inf, jnp.float32)

    for r in range(_RPW):
        init = tuple(
            (neg_inf, jnp.zeros((_LANES,), jnp.int32), lane + k * _LANES)
            for k in range(_NCHAIN)
        )

        @pl.loop(0, _CPR, init_carry=init, step=2)
        def chunk_loop(c, carry):
            for b in range(2):
                g = r * _CPR + c + b
                pltpu.make_async_copy(
                    x_hbm.at[pl.ds(base, _CHUNK)], bufs[b], sems[b]
                ).wait()

                @pl.loop(0, _CHUNK, init_carry=carry,
                         step=_LANES * _NCHAIN)
                def inner(off, ic):
                    nxt = []
                    for k in range(_NCHAIN):
                        m, mi, idx = ic[k]
                        v = bufs[b][pl.ds(off + k * _LANES, _LANES)]
                        p = v > m
                        nxt.append((
                            jnp.where(p, v, m),
                            jnp.where(p, idx, mi),
                            idx + _LANES * _NCHAIN,
                        ))
                    return tuple(nxt)

                carry = inner

                @pl.when(g + 2 < _NCHUNKS)
                def _():
                    start(g + 2, b)

            return carry

        m, mi, _ = chunk_loop[0]
        for k in range(1, _NCHAIN):
            bm, bmi, _ = chunk_loop[k]
            p = (bm > m) | ((bm == m) & (bmi < mi))
            m = jnp.where(p, bm, m)
            mi = jnp.where(p, bmi, mi)
        for shift in (8, 4, 2, 1):
            perm = lane ^ shift
            ov = _lane_gather(m, perm)
            oi = _lane_gather(mi, perm)
            p = (ov > m) | ((ov == m) & (oi < mi))
            m = jnp.where(p, ov, m)
            mi = jnp.where(p, oi, mi)
        res = jnp.where(lane == r, mi, res)

    res_v[...] = res
    pltpu.sync_copy(res_v, out_hbm.at[wid])


@jax.jit
def _sc_argmax(flat):
    mesh = plsc.VectorSubcoreMesh(
        core_axis_name="c", subcore_axis_name="s",
        num_cores=_NC, num_subcores=_NS)
    f = pl.kernel(
        _sc_argmax_body,
        out_type=jax.ShapeDtypeStruct((_NW, _LANES), jnp.int32),
        mesh=mesh,
        scratch_types=[
            pltpu.VMEM((_CHUNK,), jnp.float32),
            pltpu.VMEM((_CHUNK,), jnp.float32),
            pltpu.VMEM((_LANES,), jnp.int32),
            pltpu.SemaphoreType.DMA,
            pltpu.SemaphoreType.DMA,
        ],
    )
    return f(flat)


def kernel(logits):
    assert logits.shape == (_BATCH, _VOCAB)
    padded = _sc_argmax(logits.reshape(-1))
    return padded[:, :_RPW].reshape(_BATCH)


# speedup vs baseline: 1.0264x; 1.0038x over previous
import functools

import jax
import jax.numpy as jnp
from jax import lax
from jax.experimental import pallas as pl
from jax.experimental.pallas import tpu as pltpu
from jax.experimental.pallas import tpu_sc as plsc

_BATCH = 128
_VOCAB = 100000
_NC = 2
_NS = 16
_NW = _NC * _NS
_RPW = _BATCH // _NW
_CHUNK = 10000
_CPR = _VOCAB // _CHUNK
_NCHUNKS = _RPW * _CPR
_LANES = 16
_NCHAIN = 5


def _lane_gather(x, idx):
    return lax.gather(
        x,
        idx[:, None],
        dimension_numbers=lax.GatherDimensionNumbers(
            offset_dims=(), collapsed_slice_dims=(0,), start_index_map=(0,)),
        slice_sizes=(1,),
        mode=lax.GatherScatterMode.PROMISE_IN_BOUNDS,
    )


def _sc_argmax_body(x_hbm, out_hbm, buf0, buf1, res_v, sem0, sem1):
    wid = lax.axis_index("s") * _NC + lax.axis_index("c")
    row0 = wid * _RPW
    bufs = (buf0, buf1)
    sems = (sem0, sem1)

    base = row0 * _VOCAB

    def start(g, b):
        pltpu.make_async_copy(
            x_hbm.at[pl.ds(base + g * _CHUNK, _CHUNK)],
            bufs[b],
            sems[b],
        ).start()

    start(0, 0)
    start(1, 1)

    lane = lax.iota(jnp.int32, _LANES)
    res = jnp.zeros((_LANES,), jnp.int32)
    neg_inf = jnp.full((_LANES,), -jnp.inf, jnp.float32)

    for r in range(_RPW):
        init = tuple(
            (neg_inf, jnp.zeros((_LANES,), jnp.int32), lane + k * _LANES)
            for k in range(_NCHAIN)
        )

        @pl.loop(0, _CPR, init_carry=init, step=2)
        def chunk_loop(c, carry):
            for b in range(2):
                g = r * _CPR + c + b
                pltpu.make_async_copy(
                    x_hbm.at[pl.ds(0, _CHUNK)], bufs[b], sems[b]
                ).wait()

                @pl.loop(0, _CHUNK, init_carry=carry,
                         step=_LANES * _NCHAIN)
                def inner(off, ic):
                    nxt = []
                    for k in range(_NCHAIN):
                        m, mi, idx = ic[k]
                        v = bufs[b][pl.ds(off + k * _LANES, _LANES)]
                        p = v > m
                        nxt.append((
                            jnp.where(p, v, m),
                            jnp.where(p, idx, mi),
                            idx + _LANES * _NCHAIN,
                        ))
                    return tuple(nxt)

                carry = inner

                @pl.when(g + 2 < _NCHUNKS)
                def _():
                    start(g + 2, b)

            return carry

        m, mi, _ = chunk_loop[0]
        for k in range(1, _NCHAIN):
            bm, bmi, _ = chunk_loop[k]
            p = (bm > m) | ((bm == m) & (bmi < mi))
            m = jnp.where(p, bm, m)
            mi = jnp.where(p, bmi, mi)
        for shift in (8, 4, 2, 1):
            perm = lane ^ shift
            ov = _lane_gather(m, perm)
            oi = _lane_gather(mi, perm)
            p = (ov > m) | ((ov == m) & (oi < mi))
            m = jnp.where(p, ov, m)
            mi = jnp.where(p, oi, mi)
        res = jnp.where(lane == r, mi, res)

    res_v[...] = res
    pltpu.sync_copy(res_v, out_hbm.at[wid])


@jax.jit
def _sc_argmax(x):
    mesh = plsc.VectorSubcoreMesh(
        core_axis_name="c", subcore_axis_name="s",
        num_cores=_NC, num_subcores=_NS)
    f = pl.kernel(
        _sc_argmax_body,
        out_type=jax.ShapeDtypeStruct((_NW, _LANES), jnp.int32),
        mesh=mesh,
        scratch_types=[
            pltpu.VMEM((_CHUNK,), jnp.float32),
            pltpu.VMEM((_CHUNK,), jnp.float32),
            pltpu.VMEM((_LANES,), jnp.int32),
            pltpu.SemaphoreType.DMA,
            pltpu.SemaphoreType.DMA,
        ],
    )
    return f(x)


def kernel(logits):
    assert logits.shape == (_BATCH, _VOCAB)
    padded = _sc_argmax(logits.reshape(-1))
    return padded[:, :_RPW].reshape(_BATCH)


# speedup vs baseline: 1.0306x; 1.0040x over previous
import functools

import jax
import jax.numpy as jnp
from jax import lax
from jax.experimental import pallas as pl
from jax.experimental.pallas import tpu as pltpu
from jax.experimental.pallas import tpu_sc as plsc

_BATCH = 128
_VOCAB = 100000
_NC = 2
_NS = 16
_NW = _NC * _NS
_RPW = _BATCH // _NW
_CHUNK = 10000
_CPR = _VOCAB // _CHUNK
_NCHUNKS = _RPW * _CPR
_LANES = 16
_NCHAIN = 5


def _lane_gather(x, idx):
    return lax.gather(
        x,
        idx[:, None],
        dimension_numbers=lax.GatherDimensionNumbers(
            offset_dims=(), collapsed_slice_dims=(0,), start_index_map=(0,)),
        slice_sizes=(1,),
        mode=lax.GatherScatterMode.PROMISE_IN_BOUNDS,
    )


def _sc_argmax_body(x_hbm, out_hbm, buf0, buf1, res_v, sem0, sem1):
    wid = lax.axis_index("s") * _NC + lax.axis_index("c")
    row0 = wid * _RPW
    bufs = (buf0, buf1)
    sems = (sem0, sem1)

    base = row0 * _VOCAB

    def start(g, b):
        pltpu.make_async_copy(
            x_hbm.at[pl.ds(base + g * _CHUNK, _CHUNK)],
            bufs[b],
            sems[b],
        ).start()

    start(0, 0)
    start(1, 1)

    lane = lax.iota(jnp.int32, _LANES)
    res = jnp.zeros((_LANES,), jnp.int32)
    neg_inf = jnp.full((_LANES,), -jnp.inf, jnp.float32)
    zeros = jnp.zeros((_LANES,), jnp.int32)

    for r in range(_RPW):
        init = tuple((neg_inf, zeros) for _ in range(_NCHAIN))

        @pl.loop(0, _CPR, init_carry=init, step=2)
        def chunk_loop(c, carry):
            for b in range(2):
                g = r * _CPR + c + b
                pltpu.make_async_copy(
                    x_hbm.at[pl.ds(0, _CHUNK)], bufs[b], sems[b]
                ).wait()
                cbase = (c + b) * _CHUNK

                @pl.loop(0, _CHUNK, init_carry=carry,
                         step=_LANES * _NCHAIN)
                def inner(off, ic):
                    basev = jnp.full((_LANES,), cbase + off, jnp.int32)
                    nxt = []
                    for k in range(_NCHAIN):
                        m, mo = ic[k]
                        v = bufs[b][pl.ds(off + k * _LANES, _LANES)]
                        p = v > m
                        nxt.append((
                            jnp.where(p, v, m),
                            jnp.where(p, basev, mo),
                        ))
                    return tuple(nxt)

                carry = inner

                @pl.when(g + 2 < _NCHUNKS)
                def _():
                    start(g + 2, b)

            return carry

        m, mi = chunk_loop[0]
        mi = mi + lane
        for k in range(1, _NCHAIN):
            bm, bmi = chunk_loop[k]
            bmi = bmi + (lane + k * _LANES)
            p = (bm > m) | ((bm == m) & (bmi < mi))
            m = jnp.where(p, bm, m)
            mi = jnp.where(p, bmi, mi)
        for shift in (8, 4, 2, 1):
            perm = lane ^ shift
            ov = _lane_gather(m, perm)
            oi = _lane_gather(mi, perm)
            p = (ov > m) | ((ov == m) & (oi < mi))
            m = jnp.where(p, ov, m)
            mi = jnp.where(p, oi, mi)
        res = jnp.where(lane == r, mi, res)

    res_v[...] = res
    pltpu.sync_copy(res_v, out_hbm.at[wid])


@jax.jit
def _sc_argmax(x):
    mesh = plsc.VectorSubcoreMesh(
        core_axis_name="c", subcore_axis_name="s",
        num_cores=_NC, num_subcores=_NS)
    f = pl.kernel(
        _sc_argmax_body,
        out_type=jax.ShapeDtypeStruct((_NW, _LANES), jnp.int32),
        mesh=mesh,
        scratch_types=[
            pltpu.VMEM((_CHUNK,), jnp.float32),
            pltpu.VMEM((_CHUNK,), jnp.float32),
            pltpu.VMEM((_LANES,), jnp.int32),
            pltpu.SemaphoreType.DMA,
            pltpu.SemaphoreType.DMA,
        ],
    )
    return f(x)


def kernel(logits):
    assert logits.shape == (_BATCH, _VOCAB)
    padded = _sc_argmax(logits.reshape(-1))
    return padded[:, :_RPW].reshape(_BATCH)
